# Initial kernel scaffold; baseline (speedup 1.0000x reference)
import jax
import jax.numpy as jnp
from jax.experimental import pallas as pl


def kernel(x, target):
    raise NotImplementedError("write your pallas kernel here")


# speedup vs baseline: 1.8180x; 1.8180x over previous
import math

import jax
import jax.numpy as jnp
from jax.experimental import pallas as pl
from jax.experimental.pallas import tpu as pltpu

_SIZE = 100000
_SMOOTHING = 0.1
_CONF = 1.0 - _SMOOTHING
_N = 1024
_FILL = _SMOOTHING / (_SIZE - 1)
_CONST = _N * ((_SIZE - 1) * _FILL * math.log(_FILL) + _CONF * math.log(_CONF))

_ROWS_PER_BLOCK = 32
_GRID = _N // _ROWS_PER_BLOCK


def _body(tgt_ref, x_ref, o_ref, acc_ref):
    i = pl.program_id(0)

    @pl.when(i == 0)
    def _init():
        acc_ref[0] = jnp.float32(0.0)

    x = x_ref[...]
    t = tgt_ref[0, 0, :]
    cols = jax.lax.broadcasted_iota(jnp.int32, x.shape, 1)
    w = jnp.where(cols == t[:, None], jnp.float32(_CONF), jnp.float32(_FILL))
    acc_ref[0] += jnp.sum(w * x)

    @pl.when(i == _GRID - 1)
    def _fin():
        o_ref[0, 0] = jnp.float32(_CONST) - acc_ref[0]


def kernel(x, target):
    tgt3 = target.astype(jnp.int32).reshape(_GRID, 1, _ROWS_PER_BLOCK)
    out = pl.pallas_call(
        _body,
        grid=(_GRID,),
        in_specs=[
            pl.BlockSpec((1, 1, _ROWS_PER_BLOCK), lambda i: (i, 0, 0)),
            pl.BlockSpec((_ROWS_PER_BLOCK, _SIZE), lambda i: (i, 0)),
        ],
        out_specs=pl.BlockSpec(memory_space=pltpu.SMEM),
        out_shape=jax.ShapeDtypeStruct((1, 1), jnp.float32),
        scratch_shapes=[pltpu.SMEM((1,), jnp.float32)],
        compiler_params=pltpu.CompilerParams(
            dimension_semantics=("arbitrary",),
        ),
    )(tgt3, x)
    return out[0, 0]


# speedup vs baseline: 1.8851x; 1.0369x over previous
import math

import jax
import jax.numpy as jnp
from jax.experimental import pallas as pl
from jax.experimental.pallas import tpu as pltpu

_SIZE = 100000
_SMOOTHING = 0.1
_CONF = 1.0 - _SMOOTHING
_N = 1024
_FILL = _SMOOTHING / (_SIZE - 1)
_CONST = _N * ((_SIZE - 1) * _FILL * math.log(_FILL) + _CONF * math.log(_CONF))

_NSTREAM = 4
_ROWS_PER_BLOCK = 8
_GRID = _N // (_ROWS_PER_BLOCK * _NSTREAM)


def _body(*refs):
    tgt_refs = refs[:_NSTREAM]
    x_refs = refs[_NSTREAM:2 * _NSTREAM]
    o_ref = refs[2 * _NSTREAM]
    acc_ref = refs[2 * _NSTREAM + 1]
    i = pl.program_id(0)

    @pl.when(i == 0)
    def _init():
        acc_ref[0] = jnp.float32(0.0)

    part = jnp.float32(0.0)
    for k in range(_NSTREAM):
        x = x_refs[k][...]
        t = tgt_refs[k][0, 0, :]
        cols = jax.lax.broadcasted_iota(jnp.int32, x.shape, 1)
        w = jnp.where(cols == t[:, None], jnp.float32(_CONF), jnp.float32(_FILL))
        part = part + jnp.sum(w * x)
    acc_ref[0] += part

    @pl.when(i == _GRID - 1)
    def _fin():
        o_ref[0, 0] = jnp.float32(_CONST) - acc_ref[0]


def kernel(x, target):
    tgt3 = target.astype(jnp.int32).reshape(_N // _ROWS_PER_BLOCK, 1, _ROWS_PER_BLOCK)
    tgt_specs = [
        pl.BlockSpec((1, 1, _ROWS_PER_BLOCK),
                     lambda i, k=k: (k * _GRID + i, 0, 0))
        for k in range(_NSTREAM)
    ]
    x_specs = [
        pl.BlockSpec((_ROWS_PER_BLOCK, _SIZE),
                     lambda i, k=k: (k * _GRID + i, 0))
        for k in range(_NSTREAM)
    ]
    out = pl.pallas_call(
        _body,
        grid=(_GRID,),
        in_specs=tgt_specs + x_specs,
        out_specs=pl.BlockSpec(memory_space=pltpu.SMEM),
        out_shape=jax.ShapeDtypeStruct((1, 1), jnp.float32),
        scratch_shapes=[pltpu.SMEM((1,), jnp.float32)],
        compiler_params=pltpu.CompilerParams(
            dimension_semantics=("arbitrary",),
        ),
    )(*([tgt3] * _NSTREAM + [x] * _NSTREAM))
    return out[0, 0]
